# baseline (speedup 1.0000x reference)
import jax
import jax.numpy as jnp
from jax import lax
from jax.experimental import pallas as pl
from jax.experimental.pallas import tpu as pltpu
from jax.experimental.pallas import tpu_sc as plsc

N_NODES = 10000
N_EDGES = 320000
D = 128

NC = 2
NS = 16
NW = NC * NS
CHUNK = 128
CHUNKS_PER_W = 80
E_PAD = NW * CHUNKS_PER_W * CHUNK
EDGES_PER_W = CHUNKS_PER_W * CHUNK

N_PAD = 10240
RPT = N_PAD // NS
N_DUMMY = N_PAD - N_NODES

_mesh = plsc.VectorSubcoreMesh(core_axis_name="c", subcore_axis_name="s")


def _fill(buf, val):
    rows, cols = buf.shape

    @pl.loop(0, rows)
    def _r(i):
        @pl.loop(0, cols // 16)
        def _c(j):
            buf[i, pl.ds(j * 16, 16)] = jnp.full((16,), val, jnp.float32)


def _make_agg():
    def body(table, srcp, dstp, out, acc,
             srcv0, dstv0, srcv1, dstv1, rows0, rows1, sem0, sem1):
        c = lax.axis_index("c")
        s = lax.axis_index("s")
        w = s * NC + c

        _fill(rows0, 0.0)

        @pl.loop(0, RPT // CHUNK)
        def _zero(t):
            pltpu.sync_copy(rows0, acc.at[pl.ds(s * RPT + t * CHUNK, CHUNK)])

        plsc.subcore_barrier()

        base = w * EDGES_PER_W

        def load_idx(k, sv, dv):
            off = base + k * CHUNK
            pltpu.sync_copy(srcp.at[pl.ds(off, CHUNK)], sv)
            pltpu.sync_copy(dstp.at[pl.ds(off, CHUNK)], dv)

        load_idx(0, srcv0, dstv0)
        pltpu.async_copy(table.at[srcv0], rows0, sem0)
        half = CHUNKS_PER_W // 2

        @pl.loop(0, half)
        def _pair(t):
            a = 2 * t
            load_idx(a + 1, srcv1, dstv1)
            pltpu.async_copy(table.at[srcv1], rows1, sem1)
            pltpu.make_async_copy(table.at[srcv0], rows0, sem0).wait()
            pltpu.sync_copy(rows0, acc.at[dstv0], add=True)

            @pl.when(t < half - 1)
            def _next():
                load_idx(a + 2, srcv0, dstv0)
                pltpu.async_copy(table.at[srcv0], rows0, sem0)

            pltpu.make_async_copy(table.at[srcv1], rows1, sem1).wait()
            pltpu.sync_copy(rows1, acc.at[dstv1], add=True)

        plsc.subcore_barrier()

        @pl.loop(0, RPT // CHUNK)
        def _read(t):
            r = s * RPT + t * CHUNK
            pltpu.sync_copy(acc.at[pl.ds(r, CHUNK)], rows0)
            pltpu.sync_copy(rows0, out.at[pl.ds(c * N_PAD + r, CHUNK)])

    return pl.kernel(
        body,
        out_type=jax.ShapeDtypeStruct((NC * N_PAD, D), jnp.float32),
        mesh=_mesh,
        scratch_types=[
            pltpu.VMEM_SHARED((N_PAD, D), jnp.float32),
            pltpu.VMEM((CHUNK,), jnp.int32),
            pltpu.VMEM((CHUNK,), jnp.int32),
            pltpu.VMEM((CHUNK,), jnp.int32),
            pltpu.VMEM((CHUNK,), jnp.int32),
            pltpu.VMEM((CHUNK, D), jnp.float32),
            pltpu.VMEM((CHUNK, D), jnp.float32),
            pltpu.SemaphoreType.DMA,
            pltpu.SemaphoreType.DMA,
        ],
    )


def _make_deg():
    def body(dstp, out, acc, dstv, onesv, sem):
        c = lax.axis_index("c")
        s = lax.axis_index("s")
        w = s * NC + c

        _fill(onesv, 0.0)

        @pl.loop(0, RPT // CHUNK)
        def _zero(t):
            pltpu.sync_copy(onesv, acc.at[pl.ds(s * RPT + t * CHUNK, CHUNK)])

        _fill(onesv, 1.0)
        plsc.subcore_barrier()

        base = w * EDGES_PER_W

        @pl.loop(0, CHUNKS_PER_W)
        def _step(j):
            pltpu.sync_copy(dstp.at[pl.ds(base + j * CHUNK, CHUNK)], dstv)
            pltpu.sync_copy(onesv, acc.at[dstv], add=True)

        plsc.subcore_barrier()

        @pl.loop(0, RPT // CHUNK)
        def _read(t):
            r = s * RPT + t * CHUNK
            pltpu.sync_copy(acc.at[pl.ds(r, CHUNK)], onesv)
            pltpu.sync_copy(onesv, out.at[pl.ds(c * N_PAD + r, CHUNK)])
            _fill(onesv, 1.0)

    return pl.kernel(
        body,
        out_type=jax.ShapeDtypeStruct((NC * N_PAD, D), jnp.float32),
        mesh=_mesh,
        scratch_types=[
            pltpu.VMEM_SHARED((N_PAD, D), jnp.float32),
            pltpu.VMEM((CHUNK,), jnp.int32),
            pltpu.VMEM((CHUNK, D), jnp.float32),
            pltpu.SemaphoreType.DMA,
        ],
    )


_agg2 = _make_agg()
_deg = _make_deg()

BLK = 1024
_GRID = N_PAD // BLK


def _mm2_body(x_ref, ws_ref, wn_ref, b_ref, xs_ref, xn_ref):
    xb = x_ref[...]
    xs_ref[...] = (jnp.dot(xb, ws_ref[...], preferred_element_type=jnp.float32)
                   + b_ref[...])
    xn_ref[...] = jnp.dot(xb, wn_ref[...], preferred_element_type=jnp.float32)


def _combine_mm2_body(xs_ref, p_ref, dg_ref, ws_ref, wn_ref, b_ref,
                      hs_ref, hn_ref):
    dsum = dg_ref[0, :, 0:1] + dg_ref[1, :, 0:1]
    invd = 1.0 / jnp.maximum(dsum, 1.0)
    h1 = jnp.maximum(xs_ref[...] + (p_ref[0] + p_ref[1]) * invd, 0.0)
    hs_ref[...] = (jnp.dot(h1, ws_ref[...], preferred_element_type=jnp.float32)
                   + b_ref[...])
    hn_ref[...] = jnp.dot(h1, wn_ref[...], preferred_element_type=jnp.float32)


def _combine_body(hs_ref, q_ref, dg_ref, out_ref):
    dsum = dg_ref[0, :, 0:1] + dg_ref[1, :, 0:1]
    invd = 1.0 / jnp.maximum(dsum, 1.0)
    out_ref[...] = jnp.maximum(
        hs_ref[...] + (q_ref[0] + q_ref[1]) * invd, 0.0)


def _row_spec(d=D):
    return pl.BlockSpec((BLK, d), lambda i: (i, 0))


def _pair_spec(d):
    return pl.BlockSpec((2, BLK, d), lambda i: (0, i, 0))


_W_SPEC = pl.BlockSpec((D, D), lambda i: (0, 0))
_B_SPEC = pl.BlockSpec((1, D), lambda i: (0, 0))

_mm2 = pl.pallas_call(
    _mm2_body,
    grid=(_GRID,),
    in_specs=[_row_spec(), _W_SPEC, _W_SPEC, _B_SPEC],
    out_specs=[_row_spec(), _row_spec()],
    out_shape=[jax.ShapeDtypeStruct((N_PAD, D), jnp.float32)] * 2,
)

_combine_mm2 = pl.pallas_call(
    _combine_mm2_body,
    grid=(_GRID,),
    in_specs=[_row_spec(), _pair_spec(D), _pair_spec(D),
              _W_SPEC, _W_SPEC, _B_SPEC],
    out_specs=[_row_spec(), _row_spec()],
    out_shape=[jax.ShapeDtypeStruct((N_PAD, D), jnp.float32)] * 2,
)

_combine = pl.pallas_call(
    _combine_body,
    grid=(_GRID,),
    in_specs=[_row_spec(), _pair_spec(D), _pair_spec(D)],
    out_specs=_row_spec(),
    out_shape=jax.ShapeDtypeStruct((N_PAD, D), jnp.float32),
)


@jax.jit
def kernel(x, edge_index, W1_self, W1_neigh, b1, W2_self, W2_neigh, b2):
    src = edge_index[0]
    dst = edge_index[1]
    n_fill = E_PAD - N_EDGES
    src_p = jnp.concatenate([src, jnp.zeros((n_fill,), jnp.int32)])
    dst_p = jnp.concatenate(
        [dst, N_NODES + (jnp.arange(n_fill, dtype=jnp.int32) % N_DUMMY)])
    x_p = jnp.pad(x, ((0, N_PAD - N_NODES), (0, 0)))

    b1r = b1.reshape(1, D)
    b2r = b2.reshape(1, D)

    xs, xn = _mm2(x_p, W1_self, W1_neigh, b1r)
    p = _agg2(xn, src_p, dst_p).reshape(NC, N_PAD, D)
    dp = _deg(dst_p).reshape(NC, N_PAD, D)
    hs, hn = _combine_mm2(xs, p, dp, W2_self, W2_neigh, b2r)
    q = _agg2(hn, src_p, dst_p).reshape(NC, N_PAD, D)
    out = _combine(hs, q, dp)
    return out[:N_NODES]


# speedup vs baseline: 2.1752x; 2.1752x over previous
import jax
import jax.numpy as jnp
from jax import lax
from jax.experimental import pallas as pl
from jax.experimental.pallas import tpu as pltpu
from jax.experimental.pallas import tpu_sc as plsc

N_NODES = 10000
N_EDGES = 320000
D = 128

NC = 2
NS = 16
NW = NC * NS
CHUNK = 128
CHUNKS_PER_W = 80
E_PAD = NW * CHUNKS_PER_W * CHUNK
EDGES_PER_W = CHUNKS_PER_W * CHUNK

N_PAD = 10240
RPT = N_PAD // NS
N_DUMMY = N_PAD - N_NODES

_mesh = plsc.VectorSubcoreMesh(core_axis_name="c", subcore_axis_name="s")


def _fill(buf, val):
    rows, cols = buf.shape

    @pl.loop(0, rows)
    def _r(i):
        @pl.loop(0, cols // 16)
        def _c(j):
            buf[i, pl.ds(j * 16, 16)] = jnp.full((16,), val, jnp.float32)


def _make_agg():
    def body(table, srcp, dstp, out, acc,
             srcv0, dstv0, srcv1, dstv1, rows0, rows1, sem0, sem1):
        c = lax.axis_index("c")
        s = lax.axis_index("s")
        w = s * NC + c

        _fill(rows0, 0.0)

        @pl.loop(0, RPT // CHUNK)
        def _zero(t):
            pltpu.sync_copy(rows0, acc.at[pl.ds(s * RPT + t * CHUNK, CHUNK)])

        plsc.subcore_barrier()

        base = w * EDGES_PER_W

        def load_idx(k, sv, dv):
            off = base + k * CHUNK
            pltpu.sync_copy(srcp.at[pl.ds(off, CHUNK)], sv)
            pltpu.sync_copy(dstp.at[pl.ds(off, CHUNK)], dv)

        load_idx(0, srcv0, dstv0)
        pltpu.async_copy(table.at[srcv0], rows0, sem0)
        half = CHUNKS_PER_W // 2

        @pl.loop(0, half)
        def _pair(t):
            a = 2 * t
            load_idx(a + 1, srcv1, dstv1)
            pltpu.async_copy(table.at[srcv1], rows1, sem1)
            pltpu.make_async_copy(table.at[srcv0], rows0, sem0).wait()
            pltpu.sync_copy(rows0, acc.at[dstv0], add=True)

            @pl.when(t < half - 1)
            def _next():
                load_idx(a + 2, srcv0, dstv0)
                pltpu.async_copy(table.at[srcv0], rows0, sem0)

            pltpu.make_async_copy(table.at[srcv1], rows1, sem1).wait()
            pltpu.sync_copy(rows1, acc.at[dstv1], add=True)

        plsc.subcore_barrier()

        @pl.loop(0, RPT // CHUNK)
        def _read(t):
            r = s * RPT + t * CHUNK
            pltpu.sync_copy(acc.at[pl.ds(r, CHUNK)], rows0)
            pltpu.sync_copy(rows0, out.at[pl.ds(c * N_PAD + r, CHUNK)])

    return pl.kernel(
        body,
        out_type=jax.ShapeDtypeStruct((NC * N_PAD, D), jnp.float32),
        mesh=_mesh,
        scratch_types=[
            pltpu.VMEM_SHARED((N_PAD, D), jnp.float32),
            pltpu.VMEM((CHUNK,), jnp.int32),
            pltpu.VMEM((CHUNK,), jnp.int32),
            pltpu.VMEM((CHUNK,), jnp.int32),
            pltpu.VMEM((CHUNK,), jnp.int32),
            pltpu.VMEM((CHUNK, D), jnp.float32),
            pltpu.VMEM((CHUNK, D), jnp.float32),
            pltpu.SemaphoreType.DMA,
            pltpu.SemaphoreType.DMA,
        ],
    )


def _make_deg():
    def body(dstp, out, acc, dstv, onesv, sem):
        c = lax.axis_index("c")
        s = lax.axis_index("s")
        w = s * NC + c

        _fill(onesv, 0.0)

        @pl.loop(0, RPT // CHUNK)
        def _zero(t):
            pltpu.sync_copy(onesv, acc.at[pl.ds(s * RPT + t * CHUNK, CHUNK)])

        _fill(onesv, 1.0)
        plsc.subcore_barrier()

        base = w * EDGES_PER_W

        @pl.loop(0, CHUNKS_PER_W)
        def _step(j):
            pltpu.sync_copy(dstp.at[pl.ds(base + j * CHUNK, CHUNK)], dstv)
            pltpu.sync_copy(onesv, acc.at[dstv], add=True)

        plsc.subcore_barrier()

        @pl.loop(0, RPT // CHUNK)
        def _read(t):
            r = s * RPT + t * CHUNK
            pltpu.sync_copy(acc.at[pl.ds(r, CHUNK)], onesv)
            pltpu.sync_copy(onesv, out.at[pl.ds(c * N_PAD + r, CHUNK)])
            _fill(onesv, 1.0)

    return pl.kernel(
        body,
        out_type=jax.ShapeDtypeStruct((NC * N_PAD, D), jnp.float32),
        mesh=_mesh,
        scratch_types=[
            pltpu.VMEM_SHARED((N_PAD, D), jnp.float32),
            pltpu.VMEM((CHUNK,), jnp.int32),
            pltpu.VMEM((CHUNK, D), jnp.float32),
            pltpu.SemaphoreType.DMA,
        ],
    )


_agg2 = _make_agg()
_deg = _make_deg()

BLK = 1024
_GRID = N_PAD // BLK


def _mm2_body(x_ref, ws_ref, wn_ref, b_ref, xs_ref, xn_ref):
    xb = x_ref[...]
    xs_ref[...] = (jnp.dot(xb, ws_ref[...], preferred_element_type=jnp.float32)
                   + b_ref[...])
    xn_ref[...] = jnp.dot(xb, wn_ref[...], preferred_element_type=jnp.float32)


def _combine_mm2_body(xs_ref, p_ref, dg_ref, ws_ref, wn_ref, b_ref,
                      hs_ref, hn_ref):
    dsum = dg_ref[0, :, 0:1] + dg_ref[1, :, 0:1]
    invd = 1.0 / jnp.maximum(dsum, 1.0)
    h1 = jnp.maximum(xs_ref[...] + (p_ref[0] + p_ref[1]) * invd, 0.0)
    hs_ref[...] = (jnp.dot(h1, ws_ref[...], preferred_element_type=jnp.float32)
                   + b_ref[...])
    hn_ref[...] = jnp.dot(h1, wn_ref[...], preferred_element_type=jnp.float32)


def _combine_body(hs_ref, q_ref, dg_ref, out_ref):
    dsum = dg_ref[0, :, 0:1] + dg_ref[1, :, 0:1]
    invd = 1.0 / jnp.maximum(dsum, 1.0)
    out_ref[...] = jnp.maximum(
        hs_ref[...] + (q_ref[0] + q_ref[1]) * invd, 0.0)


def _row_spec(d=D):
    return pl.BlockSpec((BLK, d), lambda i: (i, 0))


def _pair_spec(d):
    return pl.BlockSpec((2, BLK, d), lambda i: (0, i, 0))


_W_SPEC = pl.BlockSpec((D, D), lambda i: (0, 0))
_B_SPEC = pl.BlockSpec((1, D), lambda i: (0, 0))

_mm2 = pl.pallas_call(
    _mm2_body,
    grid=(_GRID,),
    in_specs=[_row_spec(), _W_SPEC, _W_SPEC, _B_SPEC],
    out_specs=[_row_spec(), _row_spec()],
    out_shape=[jax.ShapeDtypeStruct((N_PAD, D), jnp.float32)] * 2,
)

_combine_mm2 = pl.pallas_call(
    _combine_mm2_body,
    grid=(_GRID,),
    in_specs=[_row_spec(), _pair_spec(D), _pair_spec(D),
              _W_SPEC, _W_SPEC, _B_SPEC],
    out_specs=[_row_spec(), _row_spec()],
    out_shape=[jax.ShapeDtypeStruct((N_PAD, D), jnp.float32)] * 2,
)

_combine = pl.pallas_call(
    _combine_body,
    grid=(_GRID,),
    in_specs=[_row_spec(), _pair_spec(D), _pair_spec(D)],
    out_specs=_row_spec(),
    out_shape=jax.ShapeDtypeStruct((N_PAD, D), jnp.float32),
)


@jax.jit
def kernel(x, edge_index, W1_self, W1_neigh, b1, W2_self, W2_neigh, b2):
    src = edge_index[0]
    dst = edge_index[1]
    n_fill = E_PAD - N_EDGES
    fill_idx = jnp.arange(n_fill, dtype=jnp.int32)
    src_p = jnp.concatenate([src, fill_idx % N_NODES])
    dst_p = jnp.concatenate([dst, N_NODES + fill_idx % N_DUMMY])
    x_p = jnp.pad(x, ((0, N_PAD - N_NODES), (0, 0)))

    b1r = b1.reshape(1, D)
    b2r = b2.reshape(1, D)

    xs, xn = _mm2(x_p, W1_self, W1_neigh, b1r)
    p = _agg2(xn, src_p, dst_p).reshape(NC, N_PAD, D)
    dp = _deg(dst_p).reshape(NC, N_PAD, D)
    hs, hn = _combine_mm2(xs, p, dp, W2_self, W2_neigh, b2r)
    q = _agg2(hn, src_p, dst_p).reshape(NC, N_PAD, D)
    out = _combine(hs, q, dp)
    return out[:N_NODES]


# speedup vs baseline: 2.3930x; 1.1002x over previous
import jax
import jax.numpy as jnp
from jax import lax
from jax.experimental import pallas as pl
from jax.experimental.pallas import tpu as pltpu
from jax.experimental.pallas import tpu_sc as plsc

N_NODES = 10000
N_EDGES = 320000
D = 128

NC = 2
NS = 16
NW = NC * NS
CHUNK = 128
CHUNKS_PER_W = 80
E_PAD = NW * CHUNKS_PER_W * CHUNK
EDGES_PER_W = CHUNKS_PER_W * CHUNK

N_PAD = 10240
RPT = N_PAD // NS
N_DUMMY = N_PAD - N_NODES

_mesh = plsc.VectorSubcoreMesh(core_axis_name="c", subcore_axis_name="s")


def _fill(buf, val):
    rows, cols = buf.shape

    @pl.loop(0, rows)
    def _r(i):
        @pl.loop(0, cols // 16)
        def _c(j):
            buf[i, pl.ds(j * 16, 16)] = jnp.full((16,), val, jnp.float32)


def _make_agg():
    def body(table, srcp, dstp, out, acc,
             srcv0, dstv0, srcv1, dstv1, rows0, rows1, sem0, sem1):
        c = lax.axis_index("c")
        s = lax.axis_index("s")
        w = s * NC + c

        _fill(rows0, 0.0)

        @pl.loop(0, RPT // CHUNK)
        def _zero(t):
            pltpu.sync_copy(rows0, acc.at[pl.ds(s * RPT + t * CHUNK, CHUNK)])

        plsc.subcore_barrier()

        base = w * EDGES_PER_W

        def load_idx(k, sv, dv):
            off = base + k * CHUNK
            pltpu.sync_copy(srcp.at[pl.ds(off, CHUNK)], sv)
            pltpu.sync_copy(dstp.at[pl.ds(off, CHUNK)], dv)

        load_idx(0, srcv0, dstv0)
        pltpu.async_copy(table.at[srcv0], rows0, sem0)
        half = CHUNKS_PER_W // 2

        @pl.loop(0, half)
        def _pair(t):
            a = 2 * t
            load_idx(a + 1, srcv1, dstv1)
            pltpu.async_copy(table.at[srcv1], rows1, sem1)
            pltpu.make_async_copy(table.at[srcv0], rows0, sem0).wait()
            pltpu.sync_copy(rows0, acc.at[dstv0], add=True)

            @pl.when(t < half - 1)
            def _next():
                load_idx(a + 2, srcv0, dstv0)
                pltpu.async_copy(table.at[srcv0], rows0, sem0)

            pltpu.make_async_copy(table.at[srcv1], rows1, sem1).wait()
            pltpu.sync_copy(rows1, acc.at[dstv1], add=True)

        plsc.subcore_barrier()

        @pl.loop(0, RPT // CHUNK)
        def _read(t):
            r = s * RPT + t * CHUNK
            pltpu.sync_copy(acc.at[pl.ds(r, CHUNK)], rows0)
            pltpu.sync_copy(rows0, out.at[pl.ds(c * N_PAD + r, CHUNK)])

    return pl.kernel(
        body,
        out_type=jax.ShapeDtypeStruct((NC * N_PAD, D), jnp.float32),
        mesh=_mesh,
        scratch_types=[
            pltpu.VMEM_SHARED((N_PAD, D), jnp.float32),
            pltpu.VMEM((CHUNK,), jnp.int32),
            pltpu.VMEM((CHUNK,), jnp.int32),
            pltpu.VMEM((CHUNK,), jnp.int32),
            pltpu.VMEM((CHUNK,), jnp.int32),
            pltpu.VMEM((CHUNK, D), jnp.float32),
            pltpu.VMEM((CHUNK, D), jnp.float32),
            pltpu.SemaphoreType.DMA,
            pltpu.SemaphoreType.DMA,
        ],
    )


def _make_deg():
    def body(dstp, out, acc, dstv0, dstv1, onesv, bounce, sem0, sem1):
        c = lax.axis_index("c")
        s = lax.axis_index("s")
        w = s * NC + c

        _fill(bounce, 0.0)

        @pl.loop(0, RPT // CHUNK)
        def _zero(t):
            pltpu.sync_copy(bounce, acc.at[pl.ds(s * RPT + t * CHUNK, CHUNK)])

        _fill(onesv, 1.0)
        plsc.subcore_barrier()

        base = w * EDGES_PER_W

        def load_dst(k, dv):
            pltpu.sync_copy(dstp.at[pl.ds(base + k * CHUNK, CHUNK)], dv)

        load_dst(0, dstv0)
        pltpu.async_copy(onesv, acc.at[dstv0], sem0, add=True)
        half = CHUNKS_PER_W // 2

        @pl.loop(0, half)
        def _pair(t):
            a = 2 * t
            load_dst(a + 1, dstv1)
            pltpu.async_copy(onesv, acc.at[dstv1], sem1, add=True)
            pltpu.make_async_copy(onesv, acc.at[dstv0], sem0).wait()

            @pl.when(t < half - 1)
            def _next():
                load_dst(a + 2, dstv0)
                pltpu.async_copy(onesv, acc.at[dstv0], sem0, add=True)

            pltpu.make_async_copy(onesv, acc.at[dstv1], sem1).wait()

        plsc.subcore_barrier()

        @pl.loop(0, RPT // CHUNK)
        def _read(t):
            r = s * RPT + t * CHUNK
            pltpu.sync_copy(acc.at[pl.ds(r, CHUNK)], bounce)
            pltpu.sync_copy(bounce, out.at[pl.ds(c * N_PAD + r, CHUNK)])

    return pl.kernel(
        body,
        out_type=jax.ShapeDtypeStruct((NC * N_PAD, D), jnp.float32),
        mesh=_mesh,
        scratch_types=[
            pltpu.VMEM_SHARED((N_PAD, D), jnp.float32),
            pltpu.VMEM((CHUNK,), jnp.int32),
            pltpu.VMEM((CHUNK,), jnp.int32),
            pltpu.VMEM((CHUNK, D), jnp.float32),
            pltpu.VMEM((CHUNK, D), jnp.float32),
            pltpu.SemaphoreType.DMA,
            pltpu.SemaphoreType.DMA,
        ],
    )


_agg2 = _make_agg()
_deg = _make_deg()

BLK = 1000
_GRID = N_NODES // BLK


def _mm2_body(x_ref, ws_ref, wn_ref, b_ref, xs_ref, xn_ref):
    xb = x_ref[...]
    xs_ref[...] = (jnp.dot(xb, ws_ref[...], preferred_element_type=jnp.float32)
                   + b_ref[...])
    xn_ref[...] = jnp.dot(xb, wn_ref[...], preferred_element_type=jnp.float32)


def _combine_mm2_body(xs_ref, p_ref, dg_ref, ws_ref, wn_ref, b_ref,
                      hs_ref, hn_ref):
    dsum = dg_ref[0, :, 0:1] + dg_ref[1, :, 0:1]
    invd = 1.0 / jnp.maximum(dsum, 1.0)
    h1 = jnp.maximum(xs_ref[...] + (p_ref[0] + p_ref[1]) * invd, 0.0)
    hs_ref[...] = (jnp.dot(h1, ws_ref[...], preferred_element_type=jnp.float32)
                   + b_ref[...])
    hn_ref[...] = jnp.dot(h1, wn_ref[...], preferred_element_type=jnp.float32)


def _combine_body(hs_ref, q_ref, dg_ref, out_ref):
    dsum = dg_ref[0, :, 0:1] + dg_ref[1, :, 0:1]
    invd = 1.0 / jnp.maximum(dsum, 1.0)
    out_ref[...] = jnp.maximum(
        hs_ref[...] + (q_ref[0] + q_ref[1]) * invd, 0.0)


def _row_spec(d=D):
    return pl.BlockSpec((BLK, d), lambda i: (i, 0))


def _pair_spec(d):
    return pl.BlockSpec((2, BLK, d), lambda i: (0, i, 0))


_W_SPEC = pl.BlockSpec((D, D), lambda i: (0, 0))
_B_SPEC = pl.BlockSpec((1, D), lambda i: (0, 0))

_mm2 = pl.pallas_call(
    _mm2_body,
    grid=(_GRID,),
    in_specs=[_row_spec(), _W_SPEC, _W_SPEC, _B_SPEC],
    out_specs=[_row_spec(), _row_spec()],
    out_shape=[jax.ShapeDtypeStruct((N_NODES, D), jnp.float32)] * 2,
)

_combine_mm2 = pl.pallas_call(
    _combine_mm2_body,
    grid=(_GRID,),
    in_specs=[_row_spec(), _pair_spec(D), _pair_spec(D),
              _W_SPEC, _W_SPEC, _B_SPEC],
    out_specs=[_row_spec(), _row_spec()],
    out_shape=[jax.ShapeDtypeStruct((N_NODES, D), jnp.float32)] * 2,
)

_combine = pl.pallas_call(
    _combine_body,
    grid=(_GRID,),
    in_specs=[_row_spec(), _pair_spec(D), _pair_spec(D)],
    out_specs=_row_spec(),
    out_shape=jax.ShapeDtypeStruct((N_NODES, D), jnp.float32),
)


@jax.jit
def kernel(x, edge_index, W1_self, W1_neigh, b1, W2_self, W2_neigh, b2):
    src = edge_index[0]
    dst = edge_index[1]
    n_fill = E_PAD - N_EDGES
    fill_idx = jnp.arange(n_fill, dtype=jnp.int32)
    src_p = jnp.concatenate([src, fill_idx % N_NODES])
    dst_p = jnp.concatenate([dst, N_NODES + fill_idx % N_DUMMY])
    b1r = b1.reshape(1, D)
    b2r = b2.reshape(1, D)

    xs, xn = _mm2(x, W1_self, W1_neigh, b1r)
    p = _agg2(xn, src_p, dst_p).reshape(NC, N_PAD, D)
    dp = _deg(dst_p).reshape(NC, N_PAD, D)
    hs, hn = _combine_mm2(xs, p, dp, W2_self, W2_neigh, b2r)
    q = _agg2(hn, src_p, dst_p).reshape(NC, N_PAD, D)
    return _combine(hs, q, dp)
